# Initial kernel scaffold; baseline (speedup 1.0000x reference)
import jax
import jax.numpy as jnp
from jax.experimental import pallas as pl


def kernel(feat0, feat1, adj0, adj1, params):
    raise NotImplementedError("write your pallas kernel here")


# speedup vs baseline: 1.2883x; 1.2883x over previous
import jax
import jax.numpy as jnp
from jax.experimental import pallas as pl

_N = 4096
_H1 = 256
_H2 = 64
_BM = 256
_NB = _N // _BM
_ALPHA = 1.0


def _dot(a, b):
    return jnp.dot(a, b, preferred_element_type=jnp.float32)


def _dot_t(a, b):
    return jax.lax.dot_general(a, b, (((1,), (1,)), ((), ())),
                               preferred_element_type=jnp.float32)


def _feat_w_kernel(feat_ref, w_ref, out_ref):
    out_ref[...] = _dot(feat_ref[...], w_ref[...])


def _gcn_l1_kernel(adj_ref, u_ref, b1_ref, w2_ref, out_ref):
    x = jax.nn.relu(_dot(adj_ref[...], u_ref[...]) + b1_ref[...])
    out_ref[...] = _dot(x, w2_ref[...])


def _gcn_l2_kernel(adj_ref, v_ref, b2_ref, out_ref):
    out_ref[...] = _dot(adj_ref[...], v_ref[...]) + b2_ref[...]


def _att_w(x, aw1, ab1, aw2t):
    t = jax.nn.relu(_dot(x, aw1) + ab1)
    return jnp.sum(t * aw2t, axis=1, keepdims=True)


def _att_combine(x0, x1, aw1, ab1, aw2t):
    w0 = _att_w(x0, aw1, ab1, aw2t)
    w1 = _att_w(x1, aw1, ab1, aw2t)
    m = jnp.maximum(w0, w1)
    e0 = jnp.exp(w0 - m)
    e1 = jnp.exp(w1 - m)
    inv = 1.0 / (e0 + e1)
    return (e0 * x0 + e1 * x1) * inv


def _z_kernel(h0_ref, h1_ref, aw1_ref, ab1_ref, aw2t_ref, fgw_ref,
              z_ref, y0_ref, y1_ref):
    h0 = h0_ref[...]
    h1 = h1_ref[...]
    z_ref[...] = _att_combine(h0, h1, aw1_ref[...], ab1_ref[...],
                              aw2t_ref[...])
    y0_ref[...] = _dot(h0, fgw_ref[...])
    y1_ref[...] = _dot(h1, fgw_ref[...])


def _cluster_q(x, c, cn2):
    d = (jnp.sum(x * x, axis=1, keepdims=True) - 2.0 * _dot_t(x, c) + cn2)
    q = 1.0 / (1.0 + jnp.maximum(d, 0.0) / _ALPHA)
    q = q ** ((_ALPHA + 1.0) / 2.0)
    return q / jnp.sum(q, axis=1, keepdims=True)


def _fused_tail_kernel(h0_ref, h1_ref, z_ref, y0_ref, y1_ref, bfg_ref,
                       aw1_ref, ab1_ref, aw2t_ref, c_ref, cn2_ref,
                       wd01_ref, bd01_ref, wd02_ref, bd02_ref,
                       wd11_ref, bd11_ref, wd12_ref, bd12_ref,
                       h_ref, adjz_ref, xz0_ref, xz1_ref, qz_ref, qh_ref):
    i = pl.program_id(0)
    row = pl.ds(i * _BM, _BM)
    h0 = h0_ref[...]
    h1 = h1_ref[...]
    z = z_ref[...]
    y0 = y0_ref[...]
    y1 = y1_ref[...]
    bfg = bfg_ref[...]
    aw1 = aw1_ref[...]
    ab1 = ab1_ref[...]
    aw2t = aw2t_ref[...]
    c = c_ref[...]
    cn2 = cn2_ref[...]

    h0r = h0_ref[row, :]
    h1r = h1_ref[row, :]
    zr = z_ref[row, :]

    a0 = jax.nn.sigmoid(_dot_t(h0r, h0))
    hall0 = _dot(a0, y0) + y1_ref[row, :] + bfg
    a1 = jax.nn.sigmoid(_dot_t(h1r, h1))
    hall1 = y0_ref[row, :] + _dot(a1, y1) + bfg

    hr = _att_combine(hall0, hall1, aw1, ab1, aw2t)
    h_ref[...] = hr

    adjz_ref[...] = jax.nn.sigmoid(_dot_t(zr, z))

    t0 = jax.nn.relu(_dot(zr, wd01_ref[...]) + bd01_ref[...])
    xz0_ref[...] = _dot(t0, wd02_ref[...]) + bd02_ref[...]
    t1 = jax.nn.relu(_dot(zr, wd11_ref[...]) + bd11_ref[...])
    xz1_ref[...] = _dot(t1, wd12_ref[...]) + bd12_ref[...]

    qz_ref[...] = _cluster_q(zr, c, cn2)
    qh_ref[...] = _cluster_q(hr, c, cn2)


def _full(shape):
    return pl.BlockSpec(shape, lambda i: tuple(0 for _ in shape))


def _rows(cols, bm=_BM):
    return pl.BlockSpec((bm, cols), lambda i: (i, 0))


def kernel(feat0, feat1, adj0, adj1, params):
    enc = params["enc"]
    dec = params["dec"]
    fgw, fgb = params["fg"]
    aw1, ab1, aw2 = params["att"]
    c = params["cluster"]

    def row2(b):
        return b.reshape(1, -1)

    hidden = []
    for v, (feat, adj) in enumerate(((feat0, adj0), (feat1, adj1))):
        (w1, b1), (w2, b2) = enc[v]
        din = feat.shape[1]
        u = pl.pallas_call(
            _feat_w_kernel,
            grid=(_NB,),
            in_specs=[_rows(din), _full((din, _H1))],
            out_specs=_rows(_H1),
            out_shape=jax.ShapeDtypeStruct((_N, _H1), jnp.float32),
        )(feat, w1)
        vmat = pl.pallas_call(
            _gcn_l1_kernel,
            grid=(_NB,),
            in_specs=[_rows(_N), _full((_N, _H1)), _full((1, _H1)),
                      _full((_H1, _H2))],
            out_specs=_rows(_H2),
            out_shape=jax.ShapeDtypeStruct((_N, _H2), jnp.float32),
        )(adj, u, row2(b1), w2)
        hp = pl.pallas_call(
            _gcn_l2_kernel,
            grid=(_NB,),
            in_specs=[_rows(_N), _full((_N, _H2)), _full((1, _H2))],
            out_specs=_rows(_H2),
            out_shape=jax.ShapeDtypeStruct((_N, _H2), jnp.float32),
        )(adj, vmat, row2(b2))
        hidden.append(hp)

    h0, h1 = hidden
    aw2t = aw2.reshape(1, _H2)
    z, y0, y1 = pl.pallas_call(
        _z_kernel,
        out_shape=[jax.ShapeDtypeStruct((_N, _H2), jnp.float32)] * 3,
    )(h0, h1, aw1, row2(ab1), aw2t, fgw)

    (wd01, bd01), (wd02, bd02) = dec[0]
    (wd11, bd11), (wd12, bd12) = dec[1]
    dout = wd02.shape[1]
    cn2 = jnp.sum(c * c, axis=1).reshape(1, -1)

    h, adjz, xz0, xz1, qz, qh = pl.pallas_call(
        _fused_tail_kernel,
        grid=(_NB,),
        in_specs=[_full((_N, _H2))] * 5 + [
            _full((1, _H2)), _full((_H2, _H2)), _full((1, _H2)),
            _full((1, _H2)), _full(c.shape), _full((1, c.shape[0])),
            _full(wd01.shape), _full((1, bd01.shape[0])),
            _full(wd02.shape), _full((1, bd02.shape[0])),
            _full(wd11.shape), _full((1, bd11.shape[0])),
            _full(wd12.shape), _full((1, bd12.shape[0])),
        ],
        out_specs=[_rows(_H2), _rows(_N), _rows(dout), _rows(dout),
                   _rows(c.shape[0]), _rows(c.shape[0])],
        out_shape=[
            jax.ShapeDtypeStruct((_N, _H2), jnp.float32),
            jax.ShapeDtypeStruct((_N, _N), jnp.float32),
            jax.ShapeDtypeStruct((_N, dout), jnp.float32),
            jax.ShapeDtypeStruct((_N, dout), jnp.float32),
            jax.ShapeDtypeStruct((_N, c.shape[0]), jnp.float32),
            jax.ShapeDtypeStruct((_N, c.shape[0]), jnp.float32),
        ],
    )(h0, h1, z, y0, y1, row2(fgb), aw1, row2(ab1), aw2t, c, cn2,
      wd01, row2(bd01), wd02, row2(bd02), wd11, row2(bd11), wd12, row2(bd12))

    return (h, z, adjz, xz0, xz1, qz, qh)
